# Initial kernel scaffold; baseline (speedup 1.0000x reference)
import jax
import jax.numpy as jnp
from jax.experimental import pallas as pl
from jax.experimental.pallas import tpu as pltpu

B = 64
N = 325
IN_DIM = 2
OUT_DIM = 1
HID = 64
ORDER = 2
HORIZON = 12
SEQ = 12
NMAT = 5

GRID = 2
BB = B // GRID


def _body(inp_ref, s0_ref, s1_ref,
          e0Wg, e0bg, e0Wc, e0bc,
          e1Wg, e1bg, e1Wc, e1bc,
          d0Wg, d0bg, d0Wc, d0bc,
          d1Wg, d1bg, d1Wc, d1bc,
          pW, pb, out_ref):
    s0 = s0_ref[...]
    s1 = s1_ref[...]
    M = jnp.concatenate(
        [s0, jnp.dot(s0, s0, preferred_element_type=jnp.float32),
         s1, jnp.dot(s1, s1, preferred_element_type=jnp.float32)], axis=0)

    def gconv(x, h, W_ref, b_ref, cin, odim):
        c = cin + HID
        x0 = jnp.concatenate([x, h], axis=2)
        y = jnp.dot(M, x0.reshape(N, BB * c),
                    preferred_element_type=jnp.float32)
        W = W_ref[...]
        acc = jnp.dot(x0.reshape(N * BB, c), W[0],
                      preferred_element_type=jnp.float32)
        for k in range(4):
            tk = y[k * N:(k + 1) * N].reshape(N * BB, c)
            acc = acc + jnp.dot(tk, W[k + 1],
                                preferred_element_type=jnp.float32)
        acc = acc + b_ref[...]
        return acc.reshape(N, BB, odim)

    def cell(x, h, Wg, bg, Wc, bc, cin):
        ru = jax.nn.sigmoid(gconv(x, h, Wg, bg, cin, 2 * HID))
        r = ru[:, :, :HID]
        u = ru[:, :, HID:]
        cc = jnp.tanh(gconv(x, r * h, Wc, bc, cin, HID))
        return u * h + (1.0 - u) * cc

    h0 = jnp.zeros((N, BB, HID), jnp.float32)
    h1 = jnp.zeros((N, BB, HID), jnp.float32)

    def enc_step(t, hs):
        h0, h1 = hs
        x = inp_ref[t].reshape(N, BB, IN_DIM)
        h0n = cell(x, h0, e0Wg, e0bg, e0Wc, e0bc, IN_DIM)
        h1n = cell(h0n, h1, e1Wg, e1bg, e1Wc, e1bc, HID)
        return (h0n, h1n)

    h0, h1 = jax.lax.fori_loop(0, SEQ, enc_step, (h0, h1))

    def dec_step(t, carry):
        h0, h1, x = carry
        h0n = cell(x.reshape(N, BB, OUT_DIM), h0, d0Wg, d0bg, d0Wc, d0bc,
                   OUT_DIM)
        h1n = cell(h0n, h1, d1Wg, d1bg, d1Wc, d1bc, HID)
        y = jax.lax.dot_general(h1n.reshape(N * BB, HID), pW[...],
                                (((1,), (1,)), ((), ())),
                                preferred_element_type=jnp.float32)
        y = (y + pb[...]).reshape(N, BB)
        out_ref[t] = y
        return (h0n, h1n, y)

    jax.lax.fori_loop(0, HORIZON, dec_step,
                      (h0, h1, jnp.zeros((N, BB), jnp.float32)))


def _reorder(W, c):
    return W.reshape(c, NMAT, W.shape[1]).transpose(1, 0, 2)


def kernel(inputs, s0, s1, enc0_Wg, enc0_bg, enc0_Wc, enc0_bc,
           enc1_Wg, enc1_bg, enc1_Wc, enc1_bc,
           dec0_Wg, dec0_bg, dec0_Wc, dec0_bc,
           dec1_Wg, dec1_bg, dec1_Wc, dec1_bc, proj_W, proj_b):
    inp = inputs.transpose(3, 2, 0, 1).reshape(SEQ, N, B * IN_DIM)

    c0e, c1 = IN_DIM + HID, 2 * HID
    c0d = OUT_DIM + HID
    args = (
        inp, s0, s1,
        _reorder(enc0_Wg, c0e), enc0_bg[None], _reorder(enc0_Wc, c0e),
        enc0_bc[None],
        _reorder(enc1_Wg, c1), enc1_bg[None], _reorder(enc1_Wc, c1),
        enc1_bc[None],
        _reorder(dec0_Wg, c0d), dec0_bg[None], _reorder(dec0_Wc, c0d),
        dec0_bc[None],
        _reorder(dec1_Wg, c1), dec1_bg[None], _reorder(dec1_Wc, c1),
        dec1_bc[None],
        proj_W, proj_b[None],
    )

    def full(a):
        return pl.BlockSpec(a.shape, lambda *_: (0,) * a.ndim)

    in_specs = [
        pl.BlockSpec((SEQ, N, BB * IN_DIM), lambda i: (0, 0, i)),
        full(s0), full(s1),
    ]
    for a in args[3:]:
        in_specs.append(full(a))

    out = pl.pallas_call(
        _body,
        grid=(GRID,),
        in_specs=in_specs,
        out_specs=pl.BlockSpec((HORIZON, N, BB), lambda i: (0, 0, i)),
        out_shape=jax.ShapeDtypeStruct((HORIZON, N, B), jnp.float32),
        compiler_params=pltpu.CompilerParams(
            dimension_semantics=("arbitrary",),
        ),
    )(*args)

    return out.transpose(2, 1, 0)[:, None, :, :]


# speedup vs baseline: 3.3614x; 3.3614x over previous
import jax
import jax.numpy as jnp
from jax.experimental import pallas as pl
from jax.experimental.pallas import tpu as pltpu

B = 64
N = 325
IN_DIM = 2
OUT_DIM = 1
HID = 64
HORIZON = 12
SEQ = 12
NMAT = 5

GRID = 4
BB = B // GRID
C = 2 * HID

_DOT = dict(preferred_element_type=jnp.float32)


def _body(inp_ref, s0_ref, s1_ref,
          e0Wg, e0bg, e0Wc, e0bc,
          e1Wg, e1bg, e1Wc, e1bc,
          d0Wg, d0bg, d0Wc, d0bc,
          d1Wg, d1bg, d1Wc, d1bc,
          pW, pb, out_ref):
    s0 = s0_ref[...]
    s1 = s1_ref[...]
    M = jnp.concatenate(
        [s0, jnp.dot(s0, s0, **_DOT), s1, jnp.dot(s1, s1, **_DOT)], axis=0)

    li = jax.lax.broadcasted_iota(jnp.int32, (BB * IN_DIM, BB, HID), 0)
    bi = jax.lax.broadcasted_iota(jnp.int32, (BB * IN_DIM, BB, HID), 1)
    ci = jax.lax.broadcasted_iota(jnp.int32, (BB * IN_DIM, BB, HID), 2)
    E_enc = ((li == bi * IN_DIM + ci) & (ci < IN_DIM)).astype(jnp.float32)
    li = jax.lax.broadcasted_iota(jnp.int32, (BB, BB, HID), 0)
    bi = jax.lax.broadcasted_iota(jnp.int32, (BB, BB, HID), 1)
    ci = jax.lax.broadcasted_iota(jnp.int32, (BB, BB, HID), 2)
    E_dec = ((li == bi) & (ci == 0)).astype(jnp.float32)

    def diffuse(v3):
        return jax.lax.dot_general(M, v3, (((1,), (0,)), ((), ())), **_DOT)

    def gconv(xh3, W, b2):
        Y = diffuse(xh3)
        acc = jnp.dot(xh3.reshape(N * BB, C), W[0], **_DOT)
        for k in range(4):
            acc = acc + jnp.dot(Y[k * N:(k + 1) * N].reshape(N * BB, C),
                                W[k + 1], **_DOT)
        return acc + b2

    def cell(x3, h3, Wg_ref, bg_ref, Wc_ref, bc_ref):
        Wg, bg = Wg_ref[...], bg_ref[...]
        Wc, bc = Wc_ref[...], bc_ref[...]
        ru3 = jax.nn.sigmoid(
            gconv(jnp.concatenate([x3, h3], axis=2), Wg, bg)
        ).reshape(N, BB, C)
        r3 = ru3[:, :, :HID]
        u3 = ru3[:, :, HID:]
        c2 = jnp.tanh(
            gconv(jnp.concatenate([x3, r3 * h3], axis=2), Wc, bc))
        return u3 * h3 + (1.0 - u3) * c2.reshape(N, BB, HID)

    h0 = jnp.zeros((N, BB, HID), jnp.float32)
    h1 = jnp.zeros((N, BB, HID), jnp.float32)

    def enc_step(t, hs):
        h0, h1 = hs
        x3 = jax.lax.dot_general(inp_ref[0, t], E_enc,
                                 (((1,), (0,)), ((), ())), **_DOT)
        h0n = cell(x3, h0, e0Wg, e0bg, e0Wc, e0bc)
        h1n = cell(h0n, h1, e1Wg, e1bg, e1Wc, e1bc)
        return (h0n, h1n)

    h0, h1 = jax.lax.fori_loop(0, SEQ, enc_step, (h0, h1))

    pw = pW[...][0]
    pb0 = pb[0, 0]

    def dec_step(t, carry):
        h0, h1, y2 = carry
        x3 = jax.lax.dot_general(y2, E_dec,
                                 (((1,), (0,)), ((), ())), **_DOT)
        h0n = cell(x3, h0, d0Wg, d0bg, d0Wc, d0bc)
        h1n = cell(h0n, h1, d1Wg, d1bg, d1Wc, d1bc)
        yn = jnp.sum(h1n * pw[None, None, :], axis=2) + pb0
        out_ref[0, t] = yn
        return (h0n, h1n, yn)

    jax.lax.fori_loop(0, HORIZON, dec_step,
                      (h0, h1, jnp.zeros((N, BB), jnp.float32)))


def _prep(W, cin):
    c = cin + HID
    od = W.shape[1]
    W3 = W.reshape(c, NMAT, od).transpose(1, 0, 2)
    return jnp.concatenate(
        [W3[:, :cin, :],
         jnp.zeros((NMAT, HID - cin, od), W.dtype),
         W3[:, cin:, :]], axis=1)


def kernel(inputs, s0, s1, enc0_Wg, enc0_bg, enc0_Wc, enc0_bc,
           enc1_Wg, enc1_bg, enc1_Wc, enc1_bc,
           dec0_Wg, dec0_bg, dec0_Wc, dec0_bc,
           dec1_Wg, dec1_bg, dec1_Wc, dec1_bc, proj_W, proj_b):
    inp = (inputs.transpose(0, 3, 2, 1)
           .reshape(GRID, BB, SEQ, N, IN_DIM)
           .transpose(0, 2, 3, 1, 4)
           .reshape(GRID, SEQ, N, BB * IN_DIM))

    args = (
        inp, s0, s1,
        _prep(enc0_Wg, IN_DIM), enc0_bg[None], _prep(enc0_Wc, IN_DIM),
        enc0_bc[None],
        _prep(enc1_Wg, HID), enc1_bg[None], _prep(enc1_Wc, HID),
        enc1_bc[None],
        _prep(dec0_Wg, OUT_DIM), dec0_bg[None], _prep(dec0_Wc, OUT_DIM),
        dec0_bc[None],
        _prep(dec1_Wg, HID), dec1_bg[None], _prep(dec1_Wc, HID),
        dec1_bc[None],
        proj_W, proj_b[None],
    )

    def full(a):
        return pl.BlockSpec(a.shape, lambda *_: (0,) * a.ndim)

    in_specs = [
        pl.BlockSpec((1, SEQ, N, BB * IN_DIM), lambda i: (i, 0, 0, 0)),
        full(s0), full(s1),
    ]
    for a in args[3:]:
        in_specs.append(full(a))

    out = pl.pallas_call(
        _body,
        grid=(GRID,),
        in_specs=in_specs,
        out_specs=pl.BlockSpec((1, HORIZON, N, BB), lambda i: (i, 0, 0, 0)),
        out_shape=jax.ShapeDtypeStruct((GRID, HORIZON, N, BB), jnp.float32),
        compiler_params=pltpu.CompilerParams(
            dimension_semantics=("arbitrary",),
        ),
    )(*args)

    return (out.transpose(0, 3, 2, 1).reshape(B, N, HORIZON))[:, None, :, :]
